# Initial kernel scaffold; baseline (speedup 1.0000x reference)
import jax
import jax.numpy as jnp
from jax.experimental import pallas as pl


def kernel(x, cu_seqlens):
    raise NotImplementedError("write your pallas kernel here")


# speedup vs baseline: 4.4203x; 4.4203x over previous
import functools

import jax
import jax.numpy as jnp
from jax import lax
from jax.experimental import pallas as pl
from jax.experimental.pallas import tpu as pltpu
from jax.experimental.pallas import tpu_sc as plsc

TOK = 32768
DIM = 256
NSEG = 16
EPS = 1e-5

NW = 32
CHUNK = TOK // NW
BLK = 64
BLKP = BLK + 8
NBLK = CHUNK // BLK
LANES = 16
NSTRIP = DIM // LANES
CU_PAD = 32


def _sc_body(x_hbm, cu_hbm, psum_hbm, psq_hbm, xbuf, asum, asq, cu_v):
    wid = lax.axis_index("s") * 2 + lax.axis_index("c")
    base = wid * CHUNK
    pltpu.sync_copy(cu_hbm, cu_v)
    zeros = jnp.zeros((LANES,), jnp.float32)

    def cu_at(i):
        return cu_v[pl.ds(i, LANES)][0]

    def zero_body(s, carry):
        for k in range(NSTRIP):
            sl = pl.ds(k * LANES, LANES)
            asum[s, sl] = zeros
            asq[s, sl] = zeros
        return carry

    lax.fori_loop(0, NSEG, zero_body, jnp.int32(0))

    limit = base + CHUNK

    def seg_body(s, carry0):
        lo = jnp.maximum(cu_at(s), base)
        hi = jnp.minimum(cu_at(s + 1), limit)
        seg_len = jnp.maximum(hi - lo, 0)
        nblk = (seg_len + BLK - 1) >> 6

        def blk_body(j, carry):
            blk_lo = lo + j * BLK
            start = jnp.minimum(jnp.bitwise_and(blk_lo, -8), TOK - BLKP)
            start = pl.multiple_of(start, 8)
            pltpu.sync_copy(x_hbm.at[pl.ds(start, BLKP)], xbuf)
            loc_lo = blk_lo - start
            loc_hi = jnp.minimum(hi, blk_lo + BLK) - start
            for k in range(NSTRIP):
                sl = pl.ds(k * LANES, LANES)

                def tok_body(t, c, sl=sl):
                    sv, qv = c
                    row = xbuf[t, sl]
                    return sv + row, qv + row * row

                sv, qv = lax.fori_loop(loc_lo, loc_hi, tok_body,
                                       (zeros, zeros))
                asum[s, sl] = asum[s, sl] + sv
                asq[s, sl] = asq[s, sl] + qv
            return carry

        lax.fori_loop(0, nblk, blk_body, jnp.int32(0))
        return carry0

    lax.fori_loop(0, NSEG, seg_body, jnp.int32(0))
    pltpu.sync_copy(asum, psum_hbm.at[wid])
    pltpu.sync_copy(asq, psq_hbm.at[wid])


_sc_partials = functools.partial(
    pl.kernel,
    out_type=(
        jax.ShapeDtypeStruct((NW, NSEG, DIM), jnp.float32),
        jax.ShapeDtypeStruct((NW, NSEG, DIM), jnp.float32),
    ),
    mesh=plsc.VectorSubcoreMesh(
        core_axis_name="c", subcore_axis_name="s",
        num_cores=2, num_subcores=16),
    scratch_types=[
        pltpu.VMEM((BLKP, DIM), jnp.float32),
        pltpu.VMEM((NSEG, DIM), jnp.float32),
        pltpu.VMEM((NSEG, DIM), jnp.float32),
        pltpu.VMEM((CU_PAD,), jnp.int32),
    ],
)(_sc_body)


TBLK = 512
GRID = TOK // TBLK


def _tc_body(cu_smem, counts_ref, x_ref, psum_ref, psq_ref, o_ref,
             mean_ref, rstd_ref):
    pid = pl.program_id(0)

    @pl.when(pid == 0)
    def _():
        s = jnp.sum(psum_ref[...], axis=0)
        q = jnp.sum(psq_ref[...], axis=0)
        cnt = counts_ref[...]
        mean = s / cnt
        var = jnp.maximum(q / cnt - mean * mean, 0.0)
        mean_ref[...] = mean
        rstd_ref[...] = lax.rsqrt(var + EPS)

    tok = pid * TBLK + lax.broadcasted_iota(jnp.int32, (TBLK, 1), 0)
    bid = jnp.zeros((TBLK, 1), jnp.int32)
    for j in range(1, NSEG):
        bid += (tok >= cu_smem[j]).astype(jnp.int32)
    seg = lax.broadcasted_iota(jnp.int32, (1, NSEG), 1)
    oh = (bid == seg).astype(jnp.float32)
    mu = jnp.dot(oh, mean_ref[...], preferred_element_type=jnp.float32)
    rs = jnp.dot(oh, rstd_ref[...], preferred_element_type=jnp.float32)
    o_ref[...] = (x_ref[...] - mu) * rs


_tc_norm = pl.pallas_call(
    _tc_body,
    grid=(GRID,),
    in_specs=[
        pl.BlockSpec(memory_space=pltpu.SMEM),
        pl.BlockSpec((NSEG, 1), lambda i: (0, 0)),
        pl.BlockSpec((TBLK, DIM), lambda i: (i, 0)),
        pl.BlockSpec((NW, NSEG, DIM), lambda i: (0, 0, 0)),
        pl.BlockSpec((NW, NSEG, DIM), lambda i: (0, 0, 0)),
    ],
    out_specs=pl.BlockSpec((TBLK, DIM), lambda i: (i, 0)),
    out_shape=jax.ShapeDtypeStruct((TOK, DIM), jnp.float32),
    scratch_shapes=[
        pltpu.VMEM((NSEG, DIM), jnp.float32),
        pltpu.VMEM((NSEG, DIM), jnp.float32),
    ],
)


def kernel(x, cu_seqlens):
    cu = cu_seqlens.astype(jnp.int32)
    cu32 = jnp.concatenate(
        [cu, jnp.full((CU_PAD - NSEG - 1,), TOK, jnp.int32)])
    psum, psq = _sc_partials(x, cu32)
    counts = jnp.maximum(
        (cu[1:] - cu[:-1]).astype(jnp.float32), 1.0).reshape(NSEG, 1)
    return _tc_norm(cu32, counts, x, psum, psq)


# speedup vs baseline: 6.4312x; 1.4549x over previous
import functools

import jax
import jax.numpy as jnp
from jax import lax
from jax.experimental import pallas as pl
from jax.experimental.pallas import tpu as pltpu
from jax.experimental.pallas import tpu_sc as plsc

TOK = 32768
DIM = 256
NSEG = 16
EPS = 1e-5

NW = 32
CHUNK = TOK // NW
BLK = 64
BLKP = BLK + 8
NBLK = CHUNK // BLK
LANES = 16
NSTRIP = DIM // LANES
CU_PAD = 32


def _sc_body(x_hbm, cu_hbm, psum_hbm, psq_hbm, xbuf, asum, asq, cu_v):
    wid = lax.axis_index("s") * 2 + lax.axis_index("c")
    base = wid * CHUNK
    pltpu.sync_copy(cu_hbm, cu_v)
    zeros = jnp.zeros((LANES,), jnp.float32)

    def cu_at(i):
        return cu_v[pl.ds(i, LANES)][0]

    def zero_body(s, carry):
        for k in range(NSTRIP):
            sl = pl.ds(k * LANES, LANES)
            asum[s, sl] = zeros
            asq[s, sl] = zeros
        return carry

    lax.fori_loop(0, NSEG, zero_body, jnp.int32(0))

    limit = base + CHUNK

    def seg_body(s, carry0):
        lo = jnp.maximum(cu_at(s), base)
        hi = jnp.minimum(cu_at(s + 1), limit)
        seg_len = jnp.maximum(hi - lo, 0)
        nblk = (seg_len + BLK - 1) >> 6

        def blk_body(j, carry):
            blk_lo = lo + j * BLK
            start = jnp.minimum(jnp.bitwise_and(blk_lo, -8), TOK - BLKP)
            start = pl.multiple_of(start, 8)
            pltpu.sync_copy(x_hbm.at[pl.ds(start, BLKP)], xbuf)
            loc_lo = blk_lo - start
            loc_hi = jnp.minimum(hi, blk_lo + BLK) - start

            def tok_body(t, c):
                out = []
                for k in range(NSTRIP):
                    row = xbuf[t, pl.ds(k * LANES, LANES)]
                    out.append(c[2 * k] + row)
                    out.append(c[2 * k + 1] + row * row)
                return tuple(out)

            acc = lax.fori_loop(loc_lo, loc_hi, tok_body,
                                (zeros,) * (2 * NSTRIP))
            for k in range(NSTRIP):
                sl = pl.ds(k * LANES, LANES)
                asum[s, sl] = asum[s, sl] + acc[2 * k]
                asq[s, sl] = asq[s, sl] + acc[2 * k + 1]
            return carry

        lax.fori_loop(0, nblk, blk_body, jnp.int32(0))
        return carry0

    lax.fori_loop(0, NSEG, seg_body, jnp.int32(0))
    pltpu.sync_copy(asum, psum_hbm.at[wid])
    pltpu.sync_copy(asq, psq_hbm.at[wid])


_sc_partials = functools.partial(
    pl.kernel,
    out_type=(
        jax.ShapeDtypeStruct((NW, NSEG, DIM), jnp.float32),
        jax.ShapeDtypeStruct((NW, NSEG, DIM), jnp.float32),
    ),
    mesh=plsc.VectorSubcoreMesh(
        core_axis_name="c", subcore_axis_name="s",
        num_cores=2, num_subcores=16),
    scratch_types=[
        pltpu.VMEM((BLKP, DIM), jnp.float32),
        pltpu.VMEM((NSEG, DIM), jnp.float32),
        pltpu.VMEM((NSEG, DIM), jnp.float32),
        pltpu.VMEM((CU_PAD,), jnp.int32),
    ],
)(_sc_body)


TBLK = 512
GRID = TOK // TBLK


def _tc_stats_body(counts_ref, psum_ref, psq_ref, mean_ref, rstd_ref):
    s = jnp.sum(psum_ref[...], axis=0)
    q = jnp.sum(psq_ref[...], axis=0)
    cnt = counts_ref[...]
    mean = s / cnt
    var = jnp.maximum(q / cnt - mean * mean, 0.0)
    mean_ref[...] = mean
    rstd_ref[...] = lax.rsqrt(var + EPS)


_tc_stats = pl.pallas_call(
    _tc_stats_body,
    out_shape=(
        jax.ShapeDtypeStruct((NSEG, DIM), jnp.float32),
        jax.ShapeDtypeStruct((NSEG, DIM), jnp.float32),
    ),
)


def _tc_body(cu_smem, x_ref, mean_ref, rstd_ref, o_ref):
    pid = pl.program_id(0)
    tok = pid * TBLK + lax.broadcasted_iota(jnp.int32, (TBLK, 1), 0)
    bid = jnp.zeros((TBLK, 1), jnp.int32)
    for j in range(1, NSEG):
        bid += (tok >= cu_smem[j]).astype(jnp.int32)
    seg = lax.broadcasted_iota(jnp.int32, (1, NSEG), 1)
    oh = (bid == seg).astype(jnp.float32)
    mu = jnp.dot(oh, mean_ref[...], preferred_element_type=jnp.float32)
    rs = jnp.dot(oh, rstd_ref[...], preferred_element_type=jnp.float32)
    o_ref[...] = (x_ref[...] - mu) * rs


_tc_norm = pl.pallas_call(
    _tc_body,
    grid=(GRID,),
    in_specs=[
        pl.BlockSpec(memory_space=pltpu.SMEM),
        pl.BlockSpec((TBLK, DIM), lambda i: (i, 0)),
        pl.BlockSpec((NSEG, DIM), lambda i: (0, 0)),
        pl.BlockSpec((NSEG, DIM), lambda i: (0, 0)),
    ],
    out_specs=pl.BlockSpec((TBLK, DIM), lambda i: (i, 0)),
    out_shape=jax.ShapeDtypeStruct((TOK, DIM), jnp.float32),
)


def kernel(x, cu_seqlens):
    cu = cu_seqlens.astype(jnp.int32)
    cu32 = jnp.concatenate(
        [cu, jnp.full((CU_PAD - NSEG - 1,), TOK, jnp.int32)])
    psum, psq = _sc_partials(x, cu32)
    counts = jnp.maximum(
        (cu[1:] - cu[:-1]).astype(jnp.float32), 1.0).reshape(NSEG, 1)
    mean, rstd = _tc_stats(counts, psum, psq)
    return _tc_norm(cu32, x, mean, rstd)


# speedup vs baseline: 8.2127x; 1.2770x over previous
import functools

import jax
import jax.numpy as jnp
from jax import lax
from jax.experimental import pallas as pl
from jax.experimental.pallas import tpu as pltpu
from jax.experimental.pallas import tpu_sc as plsc

TOK = 32768
DIM = 256
NSEG = 16
EPS = 1e-5

NW = 32
CHUNK = TOK // NW
BLK = 64
BLKP = BLK + 8
NBLK = CHUNK // BLK
LANES = 16
NSTRIP = DIM // LANES
CU_PAD = 32


def _sc_body(x_hbm, cu_hbm, psum_hbm, psq_hbm, xbuf, asum, asq, cu_v):
    wid = lax.axis_index("s") * 2 + lax.axis_index("c")
    base = wid * CHUNK
    pltpu.sync_copy(cu_hbm, cu_v)
    zeros = jnp.zeros((LANES,), jnp.float32)

    def cu_at(i):
        return cu_v[pl.ds(i, LANES)][0]

    def zero_body(s, carry):
        for k in range(NSTRIP):
            sl = pl.ds(k * LANES, LANES)
            asum[s, sl] = zeros
            asq[s, sl] = zeros
        return carry

    lax.fori_loop(0, NSEG, zero_body, jnp.int32(0))

    limit = base + CHUNK

    def seg_body(s, carry0):
        lo = jnp.maximum(cu_at(s), base)
        hi = jnp.minimum(cu_at(s + 1), limit)
        seg_len = jnp.maximum(hi - lo, 0)
        nblk = (seg_len + BLK - 1) >> 6

        def blk_body(j, carry):
            blk_lo = lo + j * BLK
            start = jnp.minimum(jnp.bitwise_and(blk_lo, -8), TOK - BLKP)
            start = pl.multiple_of(start, 8)
            pltpu.sync_copy(x_hbm.at[pl.ds(start, BLKP)], xbuf)
            loc_lo = blk_lo - start
            loc_hi = jnp.minimum(hi, blk_lo + BLK) - start

            def tok_body(t, c):
                out = []
                for k in range(NSTRIP):
                    row = xbuf[t, pl.ds(k * LANES, LANES)]
                    out.append(c[2 * k] + row)
                    out.append(c[2 * k + 1] + row * row)
                return tuple(out)

            acc = lax.fori_loop(loc_lo, loc_hi, tok_body,
                                (zeros,) * (2 * NSTRIP))
            for k in range(NSTRIP):
                sl = pl.ds(k * LANES, LANES)
                asum[s, sl] = asum[s, sl] + acc[2 * k]
                asq[s, sl] = asq[s, sl] + acc[2 * k + 1]
            return carry

        lax.fori_loop(0, nblk, blk_body, jnp.int32(0))
        return carry0

    lax.fori_loop(0, NSEG, seg_body, jnp.int32(0))
    pltpu.sync_copy(asum, psum_hbm.at[wid])
    pltpu.sync_copy(asq, psq_hbm.at[wid])


_sc_partials = functools.partial(
    pl.kernel,
    out_type=(
        jax.ShapeDtypeStruct((NW, NSEG, DIM), jnp.float32),
        jax.ShapeDtypeStruct((NW, NSEG, DIM), jnp.float32),
    ),
    mesh=plsc.VectorSubcoreMesh(
        core_axis_name="c", subcore_axis_name="s",
        num_cores=2, num_subcores=16),
    scratch_types=[
        pltpu.VMEM((BLKP, DIM), jnp.float32),
        pltpu.VMEM((NSEG, DIM), jnp.float32),
        pltpu.VMEM((NSEG, DIM), jnp.float32),
        pltpu.VMEM((CU_PAD,), jnp.int32),
    ],
)(_sc_body)


TBLK = 2048
GRID = TOK // TBLK


def _tc_body(cu_smem, counts_ref, x_ref, psum_ref, psq_ref, o_ref,
             mean_ref, rstd_ref):
    pid = pl.program_id(0)

    @pl.when(pid == 0)
    def _():
        s = jnp.sum(psum_ref[...], axis=0)
        q = jnp.sum(psq_ref[...], axis=0)
        cnt = counts_ref[...]
        mean = s / cnt
        var = jnp.maximum(q / cnt - mean * mean, 0.0)
        mean_ref[...] = mean
        rstd_ref[...] = lax.rsqrt(var + EPS)

    tok = pid * TBLK + lax.broadcasted_iota(jnp.int32, (TBLK, 1), 0)
    bid = jnp.zeros((TBLK, 1), jnp.int32)
    for j in range(1, NSEG):
        bid += (tok >= cu_smem[j]).astype(jnp.int32)
    seg = lax.broadcasted_iota(jnp.int32, (1, NSEG), 1)
    oh = (bid == seg).astype(jnp.float32)
    mu = jnp.dot(oh, mean_ref[...], preferred_element_type=jnp.float32)
    rs = jnp.dot(oh, rstd_ref[...], preferred_element_type=jnp.float32)
    o_ref[...] = (x_ref[...] - mu) * rs


_tc_norm = pl.pallas_call(
    _tc_body,
    grid=(GRID,),
    in_specs=[
        pl.BlockSpec(memory_space=pltpu.SMEM),
        pl.BlockSpec((NSEG, 1), lambda i: (0, 0)),
        pl.BlockSpec((TBLK, DIM), lambda i: (i, 0)),
        pl.BlockSpec((NW, NSEG, DIM), lambda i: (0, 0, 0)),
        pl.BlockSpec((NW, NSEG, DIM), lambda i: (0, 0, 0)),
    ],
    out_specs=pl.BlockSpec((TBLK, DIM), lambda i: (i, 0)),
    out_shape=jax.ShapeDtypeStruct((TOK, DIM), jnp.float32),
    scratch_shapes=[
        pltpu.VMEM((NSEG, DIM), jnp.float32),
        pltpu.VMEM((NSEG, DIM), jnp.float32),
    ],
)


def kernel(x, cu_seqlens):
    cu = cu_seqlens.astype(jnp.int32)
    cu32 = jnp.concatenate(
        [cu, jnp.full((CU_PAD - NSEG - 1,), TOK, jnp.int32)])
    psum, psq = _sc_partials(x, cu32)
    counts = jnp.maximum(
        (cu[1:] - cu[:-1]).astype(jnp.float32), 1.0).reshape(NSEG, 1)
    return _tc_norm(cu32, counts, x, psum, psq)


# speedup vs baseline: 8.5349x; 1.0392x over previous
import functools

import jax
import jax.numpy as jnp
from jax import lax
from jax.experimental import pallas as pl
from jax.experimental.pallas import tpu as pltpu
from jax.experimental.pallas import tpu_sc as plsc

TOK = 32768
DIM = 256
NSEG = 16
EPS = 1e-5

NW = 32
CHUNK = TOK // NW
BLK = 64
BLKP = BLK + 8
NBLK = CHUNK // BLK
LANES = 16
NSTRIP = DIM // LANES
CU_PAD = 32


def _sc_body(x_hbm, cu_hbm, psum_hbm, psq_hbm, xbuf, asum, asq, cu_v):
    wid = lax.axis_index("s") * 2 + lax.axis_index("c")
    base = wid * CHUNK
    pltpu.sync_copy(cu_hbm, cu_v)
    zeros = jnp.zeros((LANES,), jnp.float32)

    def cu_at(i):
        return cu_v[pl.ds(i, LANES)][0]

    def zero_body(s, carry):
        for k in range(NSTRIP):
            sl = pl.ds(k * LANES, LANES)
            asum[s, sl] = zeros
            asq[s, sl] = zeros
        return carry

    lax.fori_loop(0, NSEG, zero_body, jnp.int32(0))

    limit = base + CHUNK

    def seg_body(s, carry0):
        lo = jnp.maximum(cu_at(s), base)
        hi = jnp.minimum(cu_at(s + 1), limit)
        seg_len = jnp.maximum(hi - lo, 0)
        nblk = (seg_len + BLK - 1) >> 6

        def blk_body(j, carry):
            blk_lo = lo + j * BLK
            start = jnp.minimum(jnp.bitwise_and(blk_lo, -8), TOK - BLKP)
            start = pl.multiple_of(start, 8)
            pltpu.sync_copy(x_hbm.at[pl.ds(start, BLKP)], xbuf)
            loc_lo = blk_lo - start
            loc_hi = jnp.minimum(hi, blk_lo + BLK) - start

            def tok_body(t, c):
                out = []
                for k in range(NSTRIP):
                    row = xbuf[t, pl.ds(k * LANES, LANES)]
                    out.append(c[2 * k] + row)
                    out.append(c[2 * k + 1] + row * row)
                return tuple(out)

            acc = lax.fori_loop(loc_lo, loc_hi, tok_body,
                                (zeros,) * (2 * NSTRIP))
            for k in range(NSTRIP):
                sl = pl.ds(k * LANES, LANES)
                asum[s, sl] = asum[s, sl] + acc[2 * k]
                asq[s, sl] = asq[s, sl] + acc[2 * k + 1]
            return carry

        lax.fori_loop(0, nblk, blk_body, jnp.int32(0))
        return carry0

    lax.fori_loop(0, NSEG, seg_body, jnp.int32(0))
    pltpu.sync_copy(asum, psum_hbm.at[wid])
    pltpu.sync_copy(asq, psq_hbm.at[wid])


_sc_partials = functools.partial(
    pl.kernel,
    out_type=(
        jax.ShapeDtypeStruct((NW, NSEG, DIM), jnp.float32),
        jax.ShapeDtypeStruct((NW, NSEG, DIM), jnp.float32),
    ),
    mesh=plsc.VectorSubcoreMesh(
        core_axis_name="c", subcore_axis_name="s",
        num_cores=2, num_subcores=16),
    scratch_types=[
        pltpu.VMEM((BLKP, DIM), jnp.float32),
        pltpu.VMEM((NSEG, DIM), jnp.float32),
        pltpu.VMEM((NSEG, DIM), jnp.float32),
        pltpu.VMEM((CU_PAD,), jnp.int32),
    ],
)(_sc_body)


TBLK = 4096
GRID = TOK // TBLK


def _tc_body(cu_smem, counts_ref, x_ref, psum_ref, psq_ref, o_ref,
             mean_ref, rstd_ref):
    pid = pl.program_id(0)

    @pl.when(pid == 0)
    def _():
        s = jnp.sum(psum_ref[...], axis=0)
        q = jnp.sum(psq_ref[...], axis=0)
        cnt = counts_ref[...]
        mean = s / cnt
        var = jnp.maximum(q / cnt - mean * mean, 0.0)
        mean_ref[...] = mean
        rstd_ref[...] = lax.rsqrt(var + EPS)

    tok = pid * TBLK + lax.broadcasted_iota(jnp.int32, (TBLK, 1), 0)
    bid = jnp.zeros((TBLK, 1), jnp.int32)
    for j in range(1, NSEG):
        bid += (tok >= cu_smem[j]).astype(jnp.int32)
    seg = lax.broadcasted_iota(jnp.int32, (1, NSEG), 1)
    oh = (bid == seg).astype(jnp.float32)
    mu = jnp.dot(oh, mean_ref[...], preferred_element_type=jnp.float32)
    rs = jnp.dot(oh, rstd_ref[...], preferred_element_type=jnp.float32)
    o_ref[...] = (x_ref[...] - mu) * rs


_tc_norm = pl.pallas_call(
    _tc_body,
    grid=(GRID,),
    in_specs=[
        pl.BlockSpec(memory_space=pltpu.SMEM),
        pl.BlockSpec((NSEG, 1), lambda i: (0, 0)),
        pl.BlockSpec((TBLK, DIM), lambda i: (i, 0)),
        pl.BlockSpec((NW, NSEG, DIM), lambda i: (0, 0, 0)),
        pl.BlockSpec((NW, NSEG, DIM), lambda i: (0, 0, 0)),
    ],
    out_specs=pl.BlockSpec((TBLK, DIM), lambda i: (i, 0)),
    out_shape=jax.ShapeDtypeStruct((TOK, DIM), jnp.float32),
    scratch_shapes=[
        pltpu.VMEM((NSEG, DIM), jnp.float32),
        pltpu.VMEM((NSEG, DIM), jnp.float32),
    ],
)


def kernel(x, cu_seqlens):
    cu = cu_seqlens.astype(jnp.int32)
    cu32 = jnp.concatenate(
        [cu, jnp.full((CU_PAD - NSEG - 1,), TOK, jnp.int32)])
    psum, psq = _sc_partials(x, cu32)
    counts = jnp.maximum(
        (cu[1:] - cu[:-1]).astype(jnp.float32), 1.0).reshape(NSEG, 1)
    return _tc_norm(cu32, counts, x, psum, psq)
